# baseline (speedup 1.0000x reference)
import functools
import math

import jax
import jax.numpy as jnp
from jax.experimental import pallas as pl
from jax.experimental.pallas import tpu as pltpu

_E = 64
_C = 32
_D = 1024
_F = 4096
_NB = 2
_BF = _F // _NB
_BD = _D // _NB


def _ffn_kernel(x_ref, w0_ref, w1_ref, o_ref, h_scr):
    e = pl.program_id(0)
    j = pl.program_id(1)

    @pl.when(e < _E)
    def _fwd():
        x = x_ref[0, 0].astype(jnp.bfloat16)
        w0 = w0_ref[0].astype(jnp.bfloat16)
        h = jax.lax.dot_general(
            x, w0, (((1,), (1,)), ((), ())),
            preferred_element_type=jnp.float32,
        )
        h = 0.5 * h * (1.0 + jax.lax.erf(h * (1.0 / math.sqrt(2.0))))
        h_scr[e % 2, :, pl.ds(j * _BF, _BF)] = h.astype(jnp.bfloat16)

    @pl.when(e > 0)
    def _bwd():
        h_full = h_scr[(e + 1) % 2]
        w1 = w1_ref[0].astype(jnp.bfloat16)
        o_ref[0, 0] = jax.lax.dot_general(
            h_full, w1, (((1,), (1,)), ((), ())),
            preferred_element_type=jnp.float32,
        )


@functools.partial(jax.jit, static_argnames=())
def _run(inputs, W0, W1):
    g = inputs.shape[0]
    out = pl.pallas_call(
        _ffn_kernel,
        grid=(_E + 1, _NB),
        in_specs=[
            pl.BlockSpec((1, 1, _C, _D),
                         lambda e, j: (0, jnp.minimum(e, _E - 1), 0, 0)),
            pl.BlockSpec((1, _BF, _D),
                         lambda e, j: (jnp.minimum(e, _E - 1), j, 0)),
            pl.BlockSpec((1, _BD, _F),
                         lambda e, j: (jnp.clip(e - 1, 0, _E - 1), j, 0)),
        ],
        out_specs=pl.BlockSpec((1, 1, _C, _BD),
                               lambda e, j: (0, jnp.clip(e - 1, 0, _E - 1), 0, j)),
        out_shape=jax.ShapeDtypeStruct((g, _E, _C, _D), jnp.float32),
        scratch_shapes=[pltpu.VMEM((2, _C, _F), jnp.bfloat16)],
    )(inputs, W0, W1)
    return out


def kernel(output_tensor, inputs, W0, W1):
    return _run(inputs, W0, W1)


# speedup vs baseline: 1.0119x; 1.0119x over previous
import functools
import math

import jax
import jax.numpy as jnp
from jax.experimental import pallas as pl
from jax.experimental.pallas import tpu as pltpu

_E = 64
_C = 32
_D = 1024
_F = 4096
_BF = 2048
_NF = _F // _BF


def _ffn_kernel(x_ref, w0_ref, w1_ref, o_ref):
    f = pl.program_id(1)
    x = x_ref[0, 0].astype(jnp.bfloat16)
    w0 = w0_ref[0].astype(jnp.bfloat16)
    h = jax.lax.dot_general(
        x, w0, (((1,), (1,)), ((), ())),
        preferred_element_type=jnp.float32,
    )
    h = 0.5 * h * (1.0 + jax.lax.erf(h * (1.0 / math.sqrt(2.0))))
    h = h.astype(jnp.bfloat16)
    w1 = w1_ref[0].astype(jnp.bfloat16)
    part = jax.lax.dot_general(
        h, w1, (((1,), (1,)), ((), ())),
        preferred_element_type=jnp.float32,
    )

    @pl.when(f == 0)
    def _init():
        o_ref[0, 0] = part

    @pl.when(f != 0)
    def _acc():
        o_ref[0, 0] += part


@functools.partial(jax.jit, static_argnames=())
def _run(inputs, W0, W1):
    g = inputs.shape[0]
    out = pl.pallas_call(
        _ffn_kernel,
        grid=(_E, _NF),
        in_specs=[
            pl.BlockSpec((1, 1, _C, _D), lambda e, f: (0, e, 0, 0)),
            pl.BlockSpec((1, _BF, _D), lambda e, f: (e, f, 0)),
            pl.BlockSpec((1, _D, _BF), lambda e, f: (e, 0, f)),
        ],
        out_specs=pl.BlockSpec((1, 1, _C, _D), lambda e, f: (0, e, 0, 0)),
        out_shape=jax.ShapeDtypeStruct((g, _E, _C, _D), jnp.float32),
        compiler_params=pltpu.CompilerParams(
            dimension_semantics=("parallel", "arbitrary"),
        ),
    )(inputs, W0, W1)
    return out


def kernel(output_tensor, inputs, W0, W1):
    return _run(inputs, W0, W1)
